# baseline (speedup 1.0000x reference)
import functools

import jax
import jax.numpy as jnp
from jax import lax
from jax.experimental import pallas as pl
from jax.experimental.pallas import tpu as pltpu
from jax.experimental.pallas import tpu_sc as plsc

S = 7
NCELL = S * S
N_ELEM = 30
BATCH = 1024
N_BOXES = 8
NROWS = BATCH * NCELL


DLANES = NCELL * N_ELEM
DBATCH = 256
DGRID = BATCH // DBATCH


def _dense_kernel(pt_ref, out_ref, acc_ref):
    step = pl.program_id(0)

    @pl.when(step == 0)
    def _():
        acc_ref[0] = 0.0
        acc_ref[1] = 0.0

    p = pt_ref[...]
    ch = lax.rem(lax.broadcasted_iota(jnp.int32, (DLANES, DBATCH), 0), N_ELEM)
    e = jnp.where(ch >= 10, jnp.exp(p), 0.0)
    gl = lax.broadcasted_iota(jnp.int32, (DLANES, NCELL), 0) // N_ELEM
    gc = lax.broadcasted_iota(jnp.int32, (DLANES, NCELL), 1)
    g = jnp.where(gl == gc, 1.0, 0.0)
    gs = jax.lax.dot_general(g, e, (((0,), (0,)), ((), ())),
                             preferred_element_type=jnp.float32)
    a_p = jnp.sum(jnp.log(gs))
    b0_p = jnp.sum(jnp.where(ch == 10, p, 0.0))
    acc_ref[0] += a_p
    acc_ref[1] += b0_p

    @pl.when(step == DGRID - 1)
    def _():
        out_ref[0, 0] = acc_ref[0]
        out_ref[0, 1] = acc_ref[1]


def _dense_call(pred_t):
    return pl.pallas_call(
        _dense_kernel,
        grid=(DGRID,),
        in_specs=[pl.BlockSpec((DLANES, DBATCH), lambda i: (0, i))],
        out_specs=pl.BlockSpec(memory_space=pltpu.MemorySpace.SMEM),
        out_shape=jax.ShapeDtypeStruct((1, 2), jnp.float32),
        scratch_shapes=[pltpu.SMEM((2,), jnp.float32)],
    )(pred_t)


NWORK = 32
BPW = BATCH // NWORK
BOXW = BPW * N_BOXES
NLS = BPW // 16


def _sc_body(pred_hbm, tgt_hbm, out_hbm, tbuf, idx4, gbuf, vbuf, ibuf, obuf,
             sem0, sem1, sem2, sem3):
    wid = lax.axis_index("s") * 2 + lax.axis_index("c")
    iota = lax.broadcasted_iota(jnp.int32, (16,), 0)

    pltpu.sync_copy(tgt_hbm, tbuf)
    tbase = jnp.right_shift(wid, 2) * 1024 + (wid & 3) * 32

    for ls in range(NLS):
        bloc = ls * 16 + iota
        cells = []
        dxs = []
        dys = []
        wxs = []
        wys = []
        for i in range(N_BOXES):
            pbase = tbase + ls * 16 + iota + i * 128
            x1 = plsc.load_gather(tbuf, [pbase + 0 * 8192])
            y1 = plsc.load_gather(tbuf, [pbase + 1 * 8192])
            x2 = plsc.load_gather(tbuf, [pbase + 2 * 8192])
            y2 = plsc.load_gather(tbuf, [pbase + 3 * 8192])
            wx7 = (x2 - x1) * float(S)
            wy7 = (y2 - y1) * float(S)
            cxs = (x1 + x2) * (0.5 * S)
            cys = (y1 + y2) * (0.5 * S)
            tx = cxs.astype(jnp.int32)
            ty = cys.astype(jnp.int32)
            cx = tx + jnp.where(tx.astype(jnp.float32) < cxs, 1, 0)
            cy = ty + jnp.where(ty.astype(jnp.float32) < cys, 1, 0)
            ijx = cx - 1
            ijy = cy - 1
            dx = cxs - ijx.astype(jnp.float32)
            dy = cys - ijy.astype(jnp.float32)
            cell = ijy * S + ijx
            cells.append(cell)
            dxs.append(dx)
            dys.append(dy)
            wxs.append(wx7)
            wys.append(wy7)
        gbatch = (wid * BPW + ls * 16 + iota) * NCELL
        for i in range(N_BOXES):
            win = cells[i] == cells[i]
            for j in range(i + 1, N_BOXES):
                win = win & (cells[j] != cells[i])
            off = ls * 128 + i * 16
            rowid = gbatch + cells[i]
            q = jnp.right_shift(rowid * N_ELEM, 4)
            idx4[ls, pl.ds(i * 16, 16)] = q
            idx4[2 + ls, pl.ds(i * 16, 16)] = q + 1
            ibuf[pl.ds(off, 16)] = rowid
            vbuf[pl.ds(0 * BOXW + off, 16)] = dxs[i]
            vbuf[pl.ds(1 * BOXW + off, 16)] = dys[i]
            vbuf[pl.ds(2 * BOXW + off, 16)] = wxs[i]
            vbuf[pl.ds(3 * BOXW + off, 16)] = wys[i]
            vbuf[pl.ds(4 * BOXW + off, 16)] = jnp.where(win, 1.0, 0.0)

    cps = []
    for h, sem in zip(range(4), (sem0, sem1, sem2, sem3)):
        cps.append(pltpu.async_copy(
            pred_hbm.at[idx4.at[h]], gbuf.at[pl.ds(h * 128, 128)], sem))
    for c in cps:
        c.wait()

    acc_loc = jnp.zeros((16,), jnp.float32)
    acc_corr = jnp.zeros((16,), jnp.float32)
    acc_cnt = jnp.zeros((16,), jnp.float32)
    for ls in range(NLS):
        for i in range(N_BOXES):
            off = ls * 128 + i * 16
            slot = off + iota
            rid = ibuf[pl.ds(off, 16)]
            woff = rid * N_ELEM - jnp.left_shift(jnp.right_shift(rid * N_ELEM, 4), 4)
            dx = vbuf[pl.ds(0 * BOXW + off, 16)]
            dy = vbuf[pl.ds(1 * BOXW + off, 16)]
            wx7 = vbuf[pl.ds(2 * BOXW + off, 16)]
            wy7 = vbuf[pl.ds(3 * BOXW + off, 16)]
            winf = vbuf[pl.ds(4 * BOXW + off, 16)]
            cls = wy7.astype(jnp.int32)

            def ld(choff):
                t = woff + choff
                row = jnp.where(t < 16, slot, 256 + slot)
                col = jnp.where(t < 16, t, t - 16)
                return plsc.load_gather(gbuf, [row, col])

            p0 = ld(0)
            p1 = ld(1)
            p2 = ld(2)
            p3 = ld(3)
            p5 = ld(5)
            p6 = ld(6)
            p7 = ld(7)
            p8 = ld(8)
            p10 = ld(10)
            pt = ld(10 + cls)
            d0 = p0 - dx
            d1 = p1 - dy
            d2 = p2 - wx7
            d3 = p3 - wy7
            d7 = p7 - dx
            d8 = p8 - dy
            sq = (d0 * d0 + d1 * d1 + d2 * d2 + d3 * d3
                  + p5 * p5 + p6 * p6 + d7 * d7 + d8 * d8)
            acc_loc = acc_loc + winf * sq
            acc_corr = acc_corr + winf * (pt - p10)
            acc_cnt = acc_cnt + winf

    s_loc = jnp.sum(acc_loc)
    s_corr = jnp.sum(acc_corr)
    s_cnt = jnp.sum(acc_cnt)
    lanei = iota
    outv = (jnp.where(lanei == 0, s_loc, 0.0)
            + jnp.where(lanei == 1, s_corr, 0.0)
            + jnp.where(lanei == 2, s_cnt, 0.0))
    obuf[...] = outv
    pltpu.sync_copy(obuf, out_hbm.at[wid])


def _sparse_call(pred, tgt_flat):
    mesh = plsc.VectorSubcoreMesh(core_axis_name="c", subcore_axis_name="s")
    f = functools.partial(
        pl.kernel,
        mesh=mesh,
        out_type=jax.ShapeDtypeStruct((NWORK, 16), jnp.float32),
        scratch_types=[
            pltpu.VMEM((BATCH * N_BOXES * 5,), jnp.float32),
            pltpu.VMEM((4, 128), jnp.int32),
            pltpu.VMEM((2 * BOXW, 16), jnp.float32),
            pltpu.VMEM((BOXW * 5,), jnp.float32),
            pltpu.VMEM((BOXW,), jnp.int32),
            pltpu.VMEM((16,), jnp.float32),
            pltpu.SemaphoreType.DMA,
            pltpu.SemaphoreType.DMA,
            pltpu.SemaphoreType.DMA,
            pltpu.SemaphoreType.DMA,
        ],
        compiler_params=pltpu.CompilerParams(
            needs_layout_passes=False, use_tc_tiling_on_sc=False),
    )(_sc_body)
    return f(pred.reshape(NROWS * N_ELEM // 16, 16), tgt_flat)


def _combine_kernel(dense_ref, sc_ref, out_ref):
    sc = sc_ref[...]
    s_loc = jnp.sum(sc[:, 0])
    s_corr = jnp.sum(sc[:, 1])
    s_cnt = jnp.sum(sc[:, 2])
    class_loss = (dense_ref[0, 0] - dense_ref[0, 1] - s_corr) / float(NROWS)
    n = jnp.maximum(s_cnt * (2.0 * 2.0), 1.0)
    out_ref[0, 0] = class_loss + 5.0 * s_loc / n


def _combine_call(dense, sparse):
    return pl.pallas_call(
        _combine_kernel,
        in_specs=[
            pl.BlockSpec(memory_space=pltpu.MemorySpace.SMEM),
            pl.BlockSpec((NWORK, 16), lambda: (0, 0)),
        ],
        out_specs=pl.BlockSpec(memory_space=pltpu.MemorySpace.SMEM),
        out_shape=jax.ShapeDtypeStruct((1, 1), jnp.float32),
    )(dense, sparse)


@jax.jit
def kernel(prediction, target):
    tview = (target.transpose(2, 1, 0).reshape(5, 8, 8, 128)
             .transpose(0, 2, 1, 3).reshape(BATCH * N_BOXES * 5))
    sparse = _sparse_call(prediction, tview)
    dense = _dense_call(prediction.T)
    return _combine_call(dense, sparse)[0, 0]


# speedup vs baseline: 1.0787x; 1.0787x over previous
import functools

import jax
import jax.numpy as jnp
from jax import lax
from jax.experimental import pallas as pl
from jax.experimental.pallas import tpu as pltpu
from jax.experimental.pallas import tpu_sc as plsc

S = 7
NCELL = S * S
N_ELEM = 30
BATCH = 1024
N_BOXES = 8
NROWS = BATCH * NCELL


DLANES = NCELL * N_ELEM
DBATCH = 256
DGRID = BATCH // DBATCH


def _dense_kernel(pt_ref, out_ref, acc_ref):
    step = pl.program_id(0)

    @pl.when(step == 0)
    def _():
        acc_ref[0] = 0.0
        acc_ref[1] = 0.0

    p = pt_ref[...]
    ch = lax.rem(lax.broadcasted_iota(jnp.int32, (DLANES, DBATCH), 0), N_ELEM)
    e = jnp.where(ch >= 10, jnp.exp(p), 0.0)
    gl = lax.broadcasted_iota(jnp.int32, (DLANES, NCELL), 0) // N_ELEM
    gc = lax.broadcasted_iota(jnp.int32, (DLANES, NCELL), 1)
    g = jnp.where(gl == gc, 1.0, 0.0)
    gs = jax.lax.dot_general(g, e, (((0,), (0,)), ((), ())),
                             preferred_element_type=jnp.float32)
    a_p = jnp.sum(jnp.log(gs))
    b0_p = jnp.sum(jnp.where(ch == 10, p, 0.0))
    acc_ref[0] += a_p
    acc_ref[1] += b0_p

    @pl.when(step == DGRID - 1)
    def _():
        out_ref[0, 0] = acc_ref[0]
        out_ref[0, 1] = acc_ref[1]


def _dense_call(pred_t):
    return pl.pallas_call(
        _dense_kernel,
        grid=(DGRID,),
        in_specs=[pl.BlockSpec((DLANES, DBATCH), lambda i: (0, i))],
        out_specs=pl.BlockSpec(memory_space=pltpu.MemorySpace.SMEM),
        out_shape=jax.ShapeDtypeStruct((1, 2), jnp.float32),
        scratch_shapes=[pltpu.SMEM((2,), jnp.float32)],
    )(pred_t)


NWORK = 32
BPW = BATCH // NWORK
BOXW = BPW * N_BOXES
NLS = BPW // 16


def _sc_body(pred_hbm, tgt_hbm, out_hbm, tbuf, idx4, gbuf, vbuf, ibuf, obuf,
             sem0, sem1, sem2, sem3):
    wid = lax.axis_index("s") * 2 + lax.axis_index("c")
    iota = lax.broadcasted_iota(jnp.int32, (16,), 0)

    whi = jnp.right_shift(wid, 2)
    for c in range(5):
        pltpu.sync_copy(tgt_hbm.at[pl.ds(c * 8192 + whi * 1024, 1024)],
                        tbuf.at[pl.ds(c * 1024, 1024)])
    tbase = (wid & 3) * 32

    for ls in range(NLS):
        bloc = ls * 16 + iota
        cells = []
        dxs = []
        dys = []
        wxs = []
        wys = []
        for i in range(N_BOXES):
            pbase = tbase + ls * 16 + iota + i * 128
            x1 = plsc.load_gather(tbuf, [pbase + 0 * 1024])
            y1 = plsc.load_gather(tbuf, [pbase + 1 * 1024])
            x2 = plsc.load_gather(tbuf, [pbase + 2 * 1024])
            y2 = plsc.load_gather(tbuf, [pbase + 3 * 1024])
            wx7 = (x2 - x1) * float(S)
            wy7 = (y2 - y1) * float(S)
            cxs = (x1 + x2) * (0.5 * S)
            cys = (y1 + y2) * (0.5 * S)
            tx = cxs.astype(jnp.int32)
            ty = cys.astype(jnp.int32)
            cx = tx + jnp.where(tx.astype(jnp.float32) < cxs, 1, 0)
            cy = ty + jnp.where(ty.astype(jnp.float32) < cys, 1, 0)
            ijx = cx - 1
            ijy = cy - 1
            dx = cxs - ijx.astype(jnp.float32)
            dy = cys - ijy.astype(jnp.float32)
            cell = ijy * S + ijx
            cells.append(cell)
            dxs.append(dx)
            dys.append(dy)
            wxs.append(wx7)
            wys.append(wy7)
        gbatch = (wid * BPW + ls * 16 + iota) * NCELL
        for i in range(N_BOXES):
            win = cells[i] == cells[i]
            for j in range(i + 1, N_BOXES):
                win = win & (cells[j] != cells[i])
            off = ls * 128 + i * 16
            rowid = gbatch + cells[i]
            q = jnp.right_shift(rowid * N_ELEM, 4)
            idx4[ls, pl.ds(i * 16, 16)] = q
            idx4[2 + ls, pl.ds(i * 16, 16)] = q + 1
            ibuf[pl.ds(off, 16)] = rowid
            vbuf[pl.ds(0 * BOXW + off, 16)] = dxs[i]
            vbuf[pl.ds(1 * BOXW + off, 16)] = dys[i]
            vbuf[pl.ds(2 * BOXW + off, 16)] = wxs[i]
            vbuf[pl.ds(3 * BOXW + off, 16)] = wys[i]
            vbuf[pl.ds(4 * BOXW + off, 16)] = jnp.where(win, 1.0, 0.0)

    cps = []
    for h, sem in zip(range(4), (sem0, sem1, sem2, sem3)):
        cps.append(pltpu.async_copy(
            pred_hbm.at[idx4.at[h]], gbuf.at[pl.ds(h * 128, 128)], sem))
    for c in cps:
        c.wait()

    acc_loc = jnp.zeros((16,), jnp.float32)
    acc_corr = jnp.zeros((16,), jnp.float32)
    acc_cnt = jnp.zeros((16,), jnp.float32)
    for ls in range(NLS):
        for i in range(N_BOXES):
            off = ls * 128 + i * 16
            slot = off + iota
            rid = ibuf[pl.ds(off, 16)]
            woff = rid * N_ELEM - jnp.left_shift(jnp.right_shift(rid * N_ELEM, 4), 4)
            dx = vbuf[pl.ds(0 * BOXW + off, 16)]
            dy = vbuf[pl.ds(1 * BOXW + off, 16)]
            wx7 = vbuf[pl.ds(2 * BOXW + off, 16)]
            wy7 = vbuf[pl.ds(3 * BOXW + off, 16)]
            winf = vbuf[pl.ds(4 * BOXW + off, 16)]
            cls = wy7.astype(jnp.int32)

            def ld(choff):
                t = woff + choff
                row = jnp.where(t < 16, slot, 256 + slot)
                col = jnp.where(t < 16, t, t - 16)
                return plsc.load_gather(gbuf, [row, col])

            p0 = ld(0)
            p1 = ld(1)
            p2 = ld(2)
            p3 = ld(3)
            p5 = ld(5)
            p6 = ld(6)
            p7 = ld(7)
            p8 = ld(8)
            p10 = ld(10)
            pt = ld(10 + cls)
            d0 = p0 - dx
            d1 = p1 - dy
            d2 = p2 - wx7
            d3 = p3 - wy7
            d7 = p7 - dx
            d8 = p8 - dy
            sq = (d0 * d0 + d1 * d1 + d2 * d2 + d3 * d3
                  + p5 * p5 + p6 * p6 + d7 * d7 + d8 * d8)
            acc_loc = acc_loc + winf * sq
            acc_corr = acc_corr + winf * (pt - p10)
            acc_cnt = acc_cnt + winf

    s_loc = jnp.sum(acc_loc)
    s_corr = jnp.sum(acc_corr)
    s_cnt = jnp.sum(acc_cnt)
    lanei = iota
    outv = (jnp.where(lanei == 0, s_loc, 0.0)
            + jnp.where(lanei == 1, s_corr, 0.0)
            + jnp.where(lanei == 2, s_cnt, 0.0))
    obuf[...] = outv
    pltpu.sync_copy(obuf, out_hbm.at[wid])


def _sparse_call(pred, tgt_flat):
    mesh = plsc.VectorSubcoreMesh(core_axis_name="c", subcore_axis_name="s")
    f = functools.partial(
        pl.kernel,
        mesh=mesh,
        out_type=jax.ShapeDtypeStruct((NWORK, 16), jnp.float32),
        scratch_types=[
            pltpu.VMEM((5 * 1024,), jnp.float32),
            pltpu.VMEM((4, 128), jnp.int32),
            pltpu.VMEM((2 * BOXW, 16), jnp.float32),
            pltpu.VMEM((BOXW * 5,), jnp.float32),
            pltpu.VMEM((BOXW,), jnp.int32),
            pltpu.VMEM((16,), jnp.float32),
            pltpu.SemaphoreType.DMA,
            pltpu.SemaphoreType.DMA,
            pltpu.SemaphoreType.DMA,
            pltpu.SemaphoreType.DMA,
        ],
        compiler_params=pltpu.CompilerParams(
            needs_layout_passes=False, use_tc_tiling_on_sc=False),
    )(_sc_body)
    return f(pred.reshape(NROWS * N_ELEM // 16, 16), tgt_flat)


def _combine_kernel(dense_ref, sc_ref, out_ref):
    sc = sc_ref[...]
    s_loc = jnp.sum(sc[:, 0])
    s_corr = jnp.sum(sc[:, 1])
    s_cnt = jnp.sum(sc[:, 2])
    class_loss = (dense_ref[0, 0] - dense_ref[0, 1] - s_corr) / float(NROWS)
    n = jnp.maximum(s_cnt * (2.0 * 2.0), 1.0)
    out_ref[0, 0] = class_loss + 5.0 * s_loc / n


def _combine_call(dense, sparse):
    return pl.pallas_call(
        _combine_kernel,
        in_specs=[
            pl.BlockSpec(memory_space=pltpu.MemorySpace.SMEM),
            pl.BlockSpec((NWORK, 16), lambda: (0, 0)),
        ],
        out_specs=pl.BlockSpec(memory_space=pltpu.MemorySpace.SMEM),
        out_shape=jax.ShapeDtypeStruct((1, 1), jnp.float32),
    )(dense, sparse)


@jax.jit
def kernel(prediction, target):
    tview = (target.transpose(2, 1, 0).reshape(5, 8, 8, 128)
             .transpose(0, 2, 1, 3).reshape(BATCH * N_BOXES * 5))
    sparse = _sparse_call(prediction, tview)
    dense = _dense_call(prediction.T)
    return _combine_call(dense, sparse)[0, 0]


# speedup vs baseline: 1.1190x; 1.0374x over previous
import functools

import jax
import jax.numpy as jnp
from jax import lax
from jax.experimental import pallas as pl
from jax.experimental.pallas import tpu as pltpu
from jax.experimental.pallas import tpu_sc as plsc

S = 7
NCELL = S * S
N_ELEM = 30
BATCH = 1024
N_BOXES = 8
NROWS = BATCH * NCELL


DLANES = NCELL * N_ELEM
DBATCH = 256
DGRID = BATCH // DBATCH


def _dense_kernel(pt_ref, out_ref, acc_ref):
    step = pl.program_id(0)

    @pl.when(step == 0)
    def _():
        acc_ref[0] = 0.0
        acc_ref[1] = 0.0

    p = pt_ref[...]
    ch = lax.rem(lax.broadcasted_iota(jnp.int32, (DLANES, DBATCH), 0), N_ELEM)
    e = jnp.where(ch >= 10, jnp.exp(p), 0.0)
    gl = lax.broadcasted_iota(jnp.int32, (DLANES, NCELL), 0) // N_ELEM
    gc = lax.broadcasted_iota(jnp.int32, (DLANES, NCELL), 1)
    g = jnp.where(gl == gc, 1.0, 0.0)
    gs = jax.lax.dot_general(g, e, (((0,), (0,)), ((), ())),
                             preferred_element_type=jnp.float32)
    a_p = jnp.sum(jnp.log(gs))
    b0_p = jnp.sum(jnp.where(ch == 10, p, 0.0))
    acc_ref[0] += a_p
    acc_ref[1] += b0_p

    @pl.when(step == DGRID - 1)
    def _():
        out_ref[0, 0] = acc_ref[0]
        out_ref[0, 1] = acc_ref[1]


def _dense_call(pred_t):
    return pl.pallas_call(
        _dense_kernel,
        grid=(DGRID,),
        in_specs=[pl.BlockSpec((DLANES, DBATCH), lambda i: (0, i))],
        out_specs=pl.BlockSpec(memory_space=pltpu.MemorySpace.SMEM),
        out_shape=jax.ShapeDtypeStruct((1, 2), jnp.float32),
        scratch_shapes=[pltpu.SMEM((2,), jnp.float32)],
    )(pred_t)


NWORK = 32
BPW = BATCH // NWORK
BOXW = BPW * N_BOXES
NLS = BPW // 16


def _sc_body(pred_hbm, tgt_hbm, out_hbm, tbuf, idx4, gbuf, vbuf, ibuf, obuf,
             sem0, sem1, sem2, sem3):
    wid = lax.axis_index("s") * 2 + lax.axis_index("c")
    iota = lax.broadcasted_iota(jnp.int32, (16,), 0)

    whi = jnp.right_shift(wid, 2)
    pltpu.sync_copy(tgt_hbm.at[:, pl.ds(whi * 1024, 1024)], tbuf)
    tbase = (wid & 3) * 32

    for ls in range(NLS):
        bloc = ls * 16 + iota
        cells = []
        dxs = []
        dys = []
        wxs = []
        wys = []
        for i in range(N_BOXES):
            pbase = tbase + ls * 16 + iota + i * 128
            x1 = plsc.load_gather(tbuf, [iota * 0 + 0, pbase])
            y1 = plsc.load_gather(tbuf, [iota * 0 + 1, pbase])
            x2 = plsc.load_gather(tbuf, [iota * 0 + 2, pbase])
            y2 = plsc.load_gather(tbuf, [iota * 0 + 3, pbase])
            wx7 = (x2 - x1) * float(S)
            wy7 = (y2 - y1) * float(S)
            cxs = (x1 + x2) * (0.5 * S)
            cys = (y1 + y2) * (0.5 * S)
            tx = cxs.astype(jnp.int32)
            ty = cys.astype(jnp.int32)
            cx = tx + jnp.where(tx.astype(jnp.float32) < cxs, 1, 0)
            cy = ty + jnp.where(ty.astype(jnp.float32) < cys, 1, 0)
            ijx = cx - 1
            ijy = cy - 1
            dx = cxs - ijx.astype(jnp.float32)
            dy = cys - ijy.astype(jnp.float32)
            cell = ijy * S + ijx
            cells.append(cell)
            dxs.append(dx)
            dys.append(dy)
            wxs.append(wx7)
            wys.append(wy7)
        gbatch = (wid * BPW + ls * 16 + iota) * NCELL
        for i in range(N_BOXES):
            win = cells[i] == cells[i]
            for j in range(i + 1, N_BOXES):
                win = win & (cells[j] != cells[i])
            off = ls * 128 + i * 16
            rowid = gbatch + cells[i]
            q = jnp.right_shift(rowid * N_ELEM, 4)
            idx4[ls, pl.ds(i * 16, 16)] = q
            idx4[2 + ls, pl.ds(i * 16, 16)] = q + 1
            ibuf[pl.ds(off, 16)] = rowid
            vbuf[pl.ds(0 * BOXW + off, 16)] = dxs[i]
            vbuf[pl.ds(1 * BOXW + off, 16)] = dys[i]
            vbuf[pl.ds(2 * BOXW + off, 16)] = wxs[i]
            vbuf[pl.ds(3 * BOXW + off, 16)] = wys[i]
            vbuf[pl.ds(4 * BOXW + off, 16)] = jnp.where(win, 1.0, 0.0)

    cps = []
    for h, sem in zip(range(4), (sem0, sem1, sem2, sem3)):
        cps.append(pltpu.async_copy(
            pred_hbm.at[idx4.at[h]], gbuf.at[pl.ds(h * 128, 128)], sem))
    for c in cps:
        c.wait()

    acc_loc = jnp.zeros((16,), jnp.float32)
    acc_corr = jnp.zeros((16,), jnp.float32)
    acc_cnt = jnp.zeros((16,), jnp.float32)
    for ls in range(NLS):
        for i in range(N_BOXES):
            off = ls * 128 + i * 16
            slot = off + iota
            rid = ibuf[pl.ds(off, 16)]
            woff = rid * N_ELEM - jnp.left_shift(jnp.right_shift(rid * N_ELEM, 4), 4)
            dx = vbuf[pl.ds(0 * BOXW + off, 16)]
            dy = vbuf[pl.ds(1 * BOXW + off, 16)]
            wx7 = vbuf[pl.ds(2 * BOXW + off, 16)]
            wy7 = vbuf[pl.ds(3 * BOXW + off, 16)]
            winf = vbuf[pl.ds(4 * BOXW + off, 16)]
            cls = wy7.astype(jnp.int32)

            def ld(choff):
                t = woff + choff
                row = jnp.where(t < 16, slot, 256 + slot)
                col = jnp.where(t < 16, t, t - 16)
                return plsc.load_gather(gbuf, [row, col])

            p0 = ld(0)
            p1 = ld(1)
            p2 = ld(2)
            p3 = ld(3)
            p5 = ld(5)
            p6 = ld(6)
            p7 = ld(7)
            p8 = ld(8)
            p10 = ld(10)
            pt = ld(10 + cls)
            d0 = p0 - dx
            d1 = p1 - dy
            d2 = p2 - wx7
            d3 = p3 - wy7
            d7 = p7 - dx
            d8 = p8 - dy
            sq = (d0 * d0 + d1 * d1 + d2 * d2 + d3 * d3
                  + p5 * p5 + p6 * p6 + d7 * d7 + d8 * d8)
            acc_loc = acc_loc + winf * sq
            acc_corr = acc_corr + winf * (pt - p10)
            acc_cnt = acc_cnt + winf

    s_loc = jnp.sum(acc_loc)
    s_corr = jnp.sum(acc_corr)
    s_cnt = jnp.sum(acc_cnt)
    lanei = iota
    outv = (jnp.where(lanei == 0, s_loc, 0.0)
            + jnp.where(lanei == 1, s_corr, 0.0)
            + jnp.where(lanei == 2, s_cnt, 0.0))
    obuf[...] = outv
    pltpu.sync_copy(obuf, out_hbm.at[wid])


def _sparse_call(pred, tgt_flat):
    mesh = plsc.VectorSubcoreMesh(core_axis_name="c", subcore_axis_name="s")
    f = functools.partial(
        pl.kernel,
        mesh=mesh,
        out_type=jax.ShapeDtypeStruct((NWORK, 16), jnp.float32),
        scratch_types=[
            pltpu.VMEM((5, 1024), jnp.float32),
            pltpu.VMEM((4, 128), jnp.int32),
            pltpu.VMEM((2 * BOXW, 16), jnp.float32),
            pltpu.VMEM((BOXW * 5,), jnp.float32),
            pltpu.VMEM((BOXW,), jnp.int32),
            pltpu.VMEM((16,), jnp.float32),
            pltpu.SemaphoreType.DMA,
            pltpu.SemaphoreType.DMA,
            pltpu.SemaphoreType.DMA,
            pltpu.SemaphoreType.DMA,
        ],
        compiler_params=pltpu.CompilerParams(
            needs_layout_passes=False, use_tc_tiling_on_sc=False),
    )(_sc_body)
    return f(pred.reshape(NROWS * N_ELEM // 16, 16), tgt_flat)


def _combine_kernel(dense_ref, sc_ref, out_ref):
    sc = sc_ref[...]
    s_loc = jnp.sum(sc[:, 0])
    s_corr = jnp.sum(sc[:, 1])
    s_cnt = jnp.sum(sc[:, 2])
    class_loss = (dense_ref[0, 0] - dense_ref[0, 1] - s_corr) / float(NROWS)
    n = jnp.maximum(s_cnt * (2.0 * 2.0), 1.0)
    out_ref[0, 0] = class_loss + 5.0 * s_loc / n


def _combine_call(dense, sparse):
    return pl.pallas_call(
        _combine_kernel,
        in_specs=[
            pl.BlockSpec(memory_space=pltpu.MemorySpace.SMEM),
            pl.BlockSpec((NWORK, 16), lambda: (0, 0)),
        ],
        out_specs=pl.BlockSpec(memory_space=pltpu.MemorySpace.SMEM),
        out_shape=jax.ShapeDtypeStruct((1, 1), jnp.float32),
    )(dense, sparse)


@jax.jit
def kernel(prediction, target):
    tview = (target.transpose(2, 1, 0).reshape(5, 8, 8, 128)
             .transpose(0, 2, 1, 3).reshape(5, N_BOXES * BATCH))
    sparse = _sparse_call(prediction, tview)
    dense = _dense_call(prediction.T)
    return _combine_call(dense, sparse)[0, 0]


# speedup vs baseline: 1.1284x; 1.0084x over previous
import functools

import jax
import jax.numpy as jnp
from jax import lax
from jax.experimental import pallas as pl
from jax.experimental.pallas import tpu as pltpu
from jax.experimental.pallas import tpu_sc as plsc

S = 7
NCELL = S * S
N_ELEM = 30
BATCH = 1024
N_BOXES = 8
NROWS = BATCH * NCELL


DLANES = NCELL * N_ELEM
DBATCH = 256
DGRID = BATCH // DBATCH


def _dense_kernel(pt_ref, out_ref, acc_ref):
    step = pl.program_id(0)

    @pl.when(step == 0)
    def _():
        acc_ref[0] = 0.0
        acc_ref[1] = 0.0

    p = pt_ref[...]
    ch = lax.rem(lax.broadcasted_iota(jnp.int32, (DLANES, DBATCH), 0), N_ELEM)
    e = jnp.where(ch >= 10, jnp.exp(p), 0.0)
    gl = lax.broadcasted_iota(jnp.int32, (DLANES, NCELL), 0) // N_ELEM
    gc = lax.broadcasted_iota(jnp.int32, (DLANES, NCELL), 1)
    g = jnp.where(gl == gc, 1.0, 0.0)
    gs = jax.lax.dot_general(g, e, (((0,), (0,)), ((), ())),
                             preferred_element_type=jnp.float32)
    a_p = jnp.sum(jnp.log(gs))
    b0_p = jnp.sum(jnp.where(ch == 10, p, 0.0))
    acc_ref[0] += a_p
    acc_ref[1] += b0_p

    @pl.when(step == DGRID - 1)
    def _():
        out_ref[0, 0] = acc_ref[0]
        out_ref[0, 1] = acc_ref[1]


def _dense_call(pred_t):
    return pl.pallas_call(
        _dense_kernel,
        grid=(DGRID,),
        in_specs=[pl.BlockSpec((DLANES, DBATCH), lambda i: (0, i))],
        out_specs=pl.BlockSpec(memory_space=pltpu.MemorySpace.SMEM),
        out_shape=jax.ShapeDtypeStruct((1, 2), jnp.float32),
        scratch_shapes=[pltpu.SMEM((2,), jnp.float32)],
    )(pred_t)


NWORK = 32
BPW = BATCH // NWORK
BOXW = BPW * N_BOXES
NLS = BPW // 16


def _sc_body(pred_hbm, tgt_hbm, out_hbm, tbuf, idx4, gbuf, vbuf, ibuf, obuf,
             sem0, sem1, sem2, sem3):
    wid = lax.axis_index("s") * 2 + lax.axis_index("c")
    iota = lax.broadcasted_iota(jnp.int32, (16,), 0)

    pltpu.sync_copy(tgt_hbm.at[pl.ds(wid * (BOXW * 5), BOXW * 5)], tbuf)

    for ls in range(NLS):
        bloc = ls * 16 + iota
        cells = []
        dxs = []
        dys = []
        wxs = []
        wys = []
        base = bloc * (N_BOXES * 5)
        for i in range(N_BOXES):
            x1 = plsc.load_gather(tbuf, [base + (5 * i + 0)])
            y1 = plsc.load_gather(tbuf, [base + (5 * i + 1)])
            x2 = plsc.load_gather(tbuf, [base + (5 * i + 2)])
            y2 = plsc.load_gather(tbuf, [base + (5 * i + 3)])
            wx7 = (x2 - x1) * float(S)
            wy7 = (y2 - y1) * float(S)
            cxs = (x1 + x2) * (0.5 * S)
            cys = (y1 + y2) * (0.5 * S)
            tx = cxs.astype(jnp.int32)
            ty = cys.astype(jnp.int32)
            cx = tx + jnp.where(tx.astype(jnp.float32) < cxs, 1, 0)
            cy = ty + jnp.where(ty.astype(jnp.float32) < cys, 1, 0)
            ijx = cx - 1
            ijy = cy - 1
            dx = cxs - ijx.astype(jnp.float32)
            dy = cys - ijy.astype(jnp.float32)
            cell = ijy * S + ijx
            cells.append(cell)
            dxs.append(dx)
            dys.append(dy)
            wxs.append(wx7)
            wys.append(wy7)
        gbatch = (wid * BPW + ls * 16 + iota) * NCELL
        for i in range(N_BOXES):
            win = cells[i] == cells[i]
            for j in range(i + 1, N_BOXES):
                win = win & (cells[j] != cells[i])
            off = ls * 128 + i * 16
            rowid = gbatch + cells[i]
            q = jnp.right_shift(rowid * N_ELEM, 4)
            idx4[ls, pl.ds(i * 16, 16)] = q
            idx4[2 + ls, pl.ds(i * 16, 16)] = q + 1
            ibuf[pl.ds(off, 16)] = rowid
            vbuf[pl.ds(0 * BOXW + off, 16)] = dxs[i]
            vbuf[pl.ds(1 * BOXW + off, 16)] = dys[i]
            vbuf[pl.ds(2 * BOXW + off, 16)] = wxs[i]
            vbuf[pl.ds(3 * BOXW + off, 16)] = wys[i]
            vbuf[pl.ds(4 * BOXW + off, 16)] = jnp.where(win, 1.0, 0.0)

    cps = []
    for h, sem in zip(range(4), (sem0, sem1, sem2, sem3)):
        cps.append(pltpu.async_copy(
            pred_hbm.at[idx4.at[h]], gbuf.at[pl.ds(h * 128, 128)], sem))
    for c in cps:
        c.wait()

    acc_loc = jnp.zeros((16,), jnp.float32)
    acc_corr = jnp.zeros((16,), jnp.float32)
    acc_cnt = jnp.zeros((16,), jnp.float32)
    for ls in range(NLS):
        for i in range(N_BOXES):
            off = ls * 128 + i * 16
            slot = off + iota
            rid = ibuf[pl.ds(off, 16)]
            woff = rid * N_ELEM - jnp.left_shift(jnp.right_shift(rid * N_ELEM, 4), 4)
            dx = vbuf[pl.ds(0 * BOXW + off, 16)]
            dy = vbuf[pl.ds(1 * BOXW + off, 16)]
            wx7 = vbuf[pl.ds(2 * BOXW + off, 16)]
            wy7 = vbuf[pl.ds(3 * BOXW + off, 16)]
            winf = vbuf[pl.ds(4 * BOXW + off, 16)]
            cls = wy7.astype(jnp.int32)

            def ld(choff):
                t = woff + choff
                row = jnp.where(t < 16, slot, 256 + slot)
                col = jnp.where(t < 16, t, t - 16)
                return plsc.load_gather(gbuf, [row, col])

            p0 = ld(0)
            p1 = ld(1)
            p2 = ld(2)
            p3 = ld(3)
            p5 = ld(5)
            p6 = ld(6)
            p7 = ld(7)
            p8 = ld(8)
            p10 = ld(10)
            pt = ld(10 + cls)
            d0 = p0 - dx
            d1 = p1 - dy
            d2 = p2 - wx7
            d3 = p3 - wy7
            d7 = p7 - dx
            d8 = p8 - dy
            sq = (d0 * d0 + d1 * d1 + d2 * d2 + d3 * d3
                  + p5 * p5 + p6 * p6 + d7 * d7 + d8 * d8)
            acc_loc = acc_loc + winf * sq
            acc_corr = acc_corr + winf * (pt - p10)
            acc_cnt = acc_cnt + winf

    s_loc = jnp.sum(acc_loc)
    s_corr = jnp.sum(acc_corr)
    s_cnt = jnp.sum(acc_cnt)
    lanei = iota
    outv = (jnp.where(lanei == 0, s_loc, 0.0)
            + jnp.where(lanei == 1, s_corr, 0.0)
            + jnp.where(lanei == 2, s_cnt, 0.0))
    obuf[...] = outv
    pltpu.sync_copy(obuf, out_hbm.at[wid])


def _sparse_call(pred, tgt_flat):
    mesh = plsc.VectorSubcoreMesh(core_axis_name="c", subcore_axis_name="s")
    f = functools.partial(
        pl.kernel,
        mesh=mesh,
        out_type=jax.ShapeDtypeStruct((NWORK, 16), jnp.float32),
        scratch_types=[
            pltpu.VMEM((BOXW * 5,), jnp.float32),
            pltpu.VMEM((4, 128), jnp.int32),
            pltpu.VMEM((2 * BOXW, 16), jnp.float32),
            pltpu.VMEM((BOXW * 5,), jnp.float32),
            pltpu.VMEM((BOXW,), jnp.int32),
            pltpu.VMEM((16,), jnp.float32),
            pltpu.SemaphoreType.DMA,
            pltpu.SemaphoreType.DMA,
            pltpu.SemaphoreType.DMA,
            pltpu.SemaphoreType.DMA,
        ],
        compiler_params=pltpu.CompilerParams(
            needs_layout_passes=False, use_tc_tiling_on_sc=False),
    )(_sc_body)
    return f(pred.reshape(NROWS * N_ELEM // 16, 16), tgt_flat)


def _combine_kernel(dense_ref, sc_ref, out_ref):
    sc = sc_ref[...]
    s_loc = jnp.sum(sc[:, 0])
    s_corr = jnp.sum(sc[:, 1])
    s_cnt = jnp.sum(sc[:, 2])
    class_loss = (dense_ref[0, 0] - dense_ref[0, 1] - s_corr) / float(NROWS)
    n = jnp.maximum(s_cnt * (2.0 * 2.0), 1.0)
    out_ref[0, 0] = class_loss + 5.0 * s_loc / n


def _combine_call(dense, sparse):
    return pl.pallas_call(
        _combine_kernel,
        in_specs=[
            pl.BlockSpec(memory_space=pltpu.MemorySpace.SMEM),
            pl.BlockSpec((NWORK, 16), lambda: (0, 0)),
        ],
        out_specs=pl.BlockSpec(memory_space=pltpu.MemorySpace.SMEM),
        out_shape=jax.ShapeDtypeStruct((1, 1), jnp.float32),
    )(dense, sparse)


@jax.jit
def kernel(prediction, target):
    sparse = _sparse_call(prediction, target.reshape(BATCH * N_BOXES * 5))
    dense = _dense_call(prediction.T)
    return _combine_call(dense, sparse)[0, 0]


# speedup vs baseline: 1.1460x; 1.0156x over previous
import functools

import jax
import jax.numpy as jnp
from jax import lax
from jax.experimental import pallas as pl
from jax.experimental.pallas import tpu as pltpu
from jax.experimental.pallas import tpu_sc as plsc

S = 7
NCELL = S * S
N_ELEM = 30
BATCH = 1024
N_BOXES = 8
NROWS = BATCH * NCELL


DLANES = NCELL * N_ELEM
DBATCH = 512
DGRID = BATCH // DBATCH


def _dense_kernel(pt_ref, out_ref, acc_ref):
    step = pl.program_id(0)

    @pl.when(step == 0)
    def _():
        acc_ref[0] = 0.0
        acc_ref[1] = 0.0

    p = pt_ref[...]
    ch = lax.rem(lax.broadcasted_iota(jnp.int32, (DLANES, DBATCH), 0), N_ELEM)
    e = jnp.where(ch >= 10, jnp.exp(p), 0.0)
    gl = lax.broadcasted_iota(jnp.int32, (DLANES, NCELL), 0) // N_ELEM
    gc = lax.broadcasted_iota(jnp.int32, (DLANES, NCELL), 1)
    g = jnp.where(gl == gc, 1.0, 0.0)
    gs = jax.lax.dot_general(g, e, (((0,), (0,)), ((), ())),
                             preferred_element_type=jnp.float32)
    a_p = jnp.sum(jnp.log(gs))
    b0_p = jnp.sum(jnp.where(ch == 10, p, 0.0))
    acc_ref[0] += a_p
    acc_ref[1] += b0_p

    @pl.when(step == DGRID - 1)
    def _():
        out_ref[0, 0] = acc_ref[0]
        out_ref[0, 1] = acc_ref[1]


def _dense_call(pred_t):
    return pl.pallas_call(
        _dense_kernel,
        grid=(DGRID,),
        in_specs=[pl.BlockSpec((DLANES, DBATCH), lambda i: (0, i))],
        out_specs=pl.BlockSpec(memory_space=pltpu.MemorySpace.SMEM),
        out_shape=jax.ShapeDtypeStruct((1, 2), jnp.float32),
        scratch_shapes=[pltpu.SMEM((2,), jnp.float32)],
    )(pred_t)


NWORK = 32
BPW = BATCH // NWORK
BOXW = BPW * N_BOXES
NLS = BPW // 16


def _sc_body(pred_hbm, tgt_hbm, out_hbm, tbuf, idx4, gbuf, vbuf, ibuf, obuf,
             sem0, sem1, sem2, sem3):
    wid = lax.axis_index("s") * 2 + lax.axis_index("c")
    iota = lax.broadcasted_iota(jnp.int32, (16,), 0)

    pltpu.sync_copy(tgt_hbm.at[pl.ds(wid * (BOXW * 5), BOXW * 5)], tbuf)

    for ls in range(NLS):
        bloc = ls * 16 + iota
        cells = []
        dxs = []
        dys = []
        wxs = []
        wys = []
        base = bloc * (N_BOXES * 5)
        for i in range(N_BOXES):
            x1 = plsc.load_gather(tbuf, [base + (5 * i + 0)])
            y1 = plsc.load_gather(tbuf, [base + (5 * i + 1)])
            x2 = plsc.load_gather(tbuf, [base + (5 * i + 2)])
            y2 = plsc.load_gather(tbuf, [base + (5 * i + 3)])
            wx7 = (x2 - x1) * float(S)
            wy7 = (y2 - y1) * float(S)
            cxs = (x1 + x2) * (0.5 * S)
            cys = (y1 + y2) * (0.5 * S)
            tx = cxs.astype(jnp.int32)
            ty = cys.astype(jnp.int32)
            cx = tx + jnp.where(tx.astype(jnp.float32) < cxs, 1, 0)
            cy = ty + jnp.where(ty.astype(jnp.float32) < cys, 1, 0)
            ijx = cx - 1
            ijy = cy - 1
            dx = cxs - ijx.astype(jnp.float32)
            dy = cys - ijy.astype(jnp.float32)
            cell = ijy * S + ijx
            cells.append(cell)
            dxs.append(dx)
            dys.append(dy)
            wxs.append(wx7)
            wys.append(wy7)
        gbatch = (wid * BPW + ls * 16 + iota) * NCELL
        for i in range(N_BOXES):
            win = cells[i] == cells[i]
            for j in range(i + 1, N_BOXES):
                win = win & (cells[j] != cells[i])
            off = ls * 128 + i * 16
            rowid = gbatch + cells[i]
            q = jnp.right_shift(rowid * N_ELEM, 4)
            idx4[ls, pl.ds(i * 16, 16)] = q
            idx4[2 + ls, pl.ds(i * 16, 16)] = q + 1
            ibuf[pl.ds(off, 16)] = rowid
            vbuf[pl.ds(0 * BOXW + off, 16)] = dxs[i]
            vbuf[pl.ds(1 * BOXW + off, 16)] = dys[i]
            vbuf[pl.ds(2 * BOXW + off, 16)] = wxs[i]
            vbuf[pl.ds(3 * BOXW + off, 16)] = wys[i]
            vbuf[pl.ds(4 * BOXW + off, 16)] = jnp.where(win, 1.0, 0.0)

    cps = []
    for h, sem in zip(range(4), (sem0, sem1, sem2, sem3)):
        cps.append(pltpu.async_copy(
            pred_hbm.at[idx4.at[h]], gbuf.at[pl.ds(h * 128, 128)], sem))
    for c in cps:
        c.wait()

    acc_loc = jnp.zeros((16,), jnp.float32)
    acc_corr = jnp.zeros((16,), jnp.float32)
    acc_cnt = jnp.zeros((16,), jnp.float32)
    for ls in range(NLS):
        for i in range(N_BOXES):
            off = ls * 128 + i * 16
            slot = off + iota
            rid = ibuf[pl.ds(off, 16)]
            woff = rid * N_ELEM - jnp.left_shift(jnp.right_shift(rid * N_ELEM, 4), 4)
            dx = vbuf[pl.ds(0 * BOXW + off, 16)]
            dy = vbuf[pl.ds(1 * BOXW + off, 16)]
            wx7 = vbuf[pl.ds(2 * BOXW + off, 16)]
            wy7 = vbuf[pl.ds(3 * BOXW + off, 16)]
            winf = vbuf[pl.ds(4 * BOXW + off, 16)]
            cls = wy7.astype(jnp.int32)

            def ld(choff):
                t = woff + choff
                row = jnp.where(t < 16, slot, 256 + slot)
                col = jnp.where(t < 16, t, t - 16)
                return plsc.load_gather(gbuf, [row, col])

            p0 = ld(0)
            p1 = ld(1)
            p2 = ld(2)
            p3 = ld(3)
            p5 = ld(5)
            p6 = ld(6)
            p7 = ld(7)
            p8 = ld(8)
            p10 = ld(10)
            pt = ld(10 + cls)
            d0 = p0 - dx
            d1 = p1 - dy
            d2 = p2 - wx7
            d3 = p3 - wy7
            d7 = p7 - dx
            d8 = p8 - dy
            sq = (d0 * d0 + d1 * d1 + d2 * d2 + d3 * d3
                  + p5 * p5 + p6 * p6 + d7 * d7 + d8 * d8)
            acc_loc = acc_loc + winf * sq
            acc_corr = acc_corr + winf * (pt - p10)
            acc_cnt = acc_cnt + winf

    s_loc = jnp.sum(acc_loc)
    s_corr = jnp.sum(acc_corr)
    s_cnt = jnp.sum(acc_cnt)
    lanei = iota
    outv = (jnp.where(lanei == 0, s_loc, 0.0)
            + jnp.where(lanei == 1, s_corr, 0.0)
            + jnp.where(lanei == 2, s_cnt, 0.0))
    obuf[...] = outv
    pltpu.sync_copy(obuf, out_hbm.at[wid])


def _sparse_call(pred, tgt_flat):
    mesh = plsc.VectorSubcoreMesh(core_axis_name="c", subcore_axis_name="s")
    f = functools.partial(
        pl.kernel,
        mesh=mesh,
        out_type=jax.ShapeDtypeStruct((NWORK, 16), jnp.float32),
        scratch_types=[
            pltpu.VMEM((BOXW * 5,), jnp.float32),
            pltpu.VMEM((4, 128), jnp.int32),
            pltpu.VMEM((2 * BOXW, 16), jnp.float32),
            pltpu.VMEM((BOXW * 5,), jnp.float32),
            pltpu.VMEM((BOXW,), jnp.int32),
            pltpu.VMEM((16,), jnp.float32),
            pltpu.SemaphoreType.DMA,
            pltpu.SemaphoreType.DMA,
            pltpu.SemaphoreType.DMA,
            pltpu.SemaphoreType.DMA,
        ],
        compiler_params=pltpu.CompilerParams(
            needs_layout_passes=False, use_tc_tiling_on_sc=False),
    )(_sc_body)
    return f(pred.reshape(NROWS * N_ELEM // 16, 16), tgt_flat)


def _combine_kernel(dense_ref, sc_ref, out_ref):
    sc = sc_ref[...]
    s_loc = jnp.sum(sc[:, 0])
    s_corr = jnp.sum(sc[:, 1])
    s_cnt = jnp.sum(sc[:, 2])
    class_loss = (dense_ref[0, 0] - dense_ref[0, 1] - s_corr) / float(NROWS)
    n = jnp.maximum(s_cnt * (2.0 * 2.0), 1.0)
    out_ref[0, 0] = class_loss + 5.0 * s_loc / n


def _combine_call(dense, sparse):
    return pl.pallas_call(
        _combine_kernel,
        in_specs=[
            pl.BlockSpec(memory_space=pltpu.MemorySpace.SMEM),
            pl.BlockSpec((NWORK, 16), lambda: (0, 0)),
        ],
        out_specs=pl.BlockSpec(memory_space=pltpu.MemorySpace.SMEM),
        out_shape=jax.ShapeDtypeStruct((1, 1), jnp.float32),
    )(dense, sparse)


@jax.jit
def kernel(prediction, target):
    sparse = _sparse_call(prediction, target.reshape(BATCH * N_BOXES * 5))
    dense = _dense_call(prediction.T)
    return _combine_call(dense, sparse)[0, 0]
